# baseline (speedup 1.0000x reference)
import functools

import jax
import jax.numpy as jnp
from jax import lax
from jax.experimental import pallas as pl
from jax.experimental.pallas import tpu as pltpu
from jax.experimental.pallas import tpu_sc as plsc

VOCAB = 10
HIDDEN = 128
NC, NS = 2, 16
NW = NC * NS
CHUNK = 128
K = 2
ROWS = K * CHUNK
NBUF = 3


@functools.partial(jax.jit, static_argnames=("nidx",))
def _emb_lookup(idx, table, nidx):
    nchunks = nidx // K

    @functools.partial(
        pl.kernel,
        out_type=jax.ShapeDtypeStruct((NW * nidx * CHUNK, HIDDEN), jnp.float32),
        mesh=plsc.VectorSubcoreMesh(core_axis_name="c", subcore_axis_name="s"),
        scratch_types=[
            pltpu.VMEM((nidx, CHUNK), jnp.int32),
            pltpu.VMEM((NBUF, ROWS, HIDDEN), jnp.float32),
            pltpu.VMEM_SHARED((VOCAB, HIDDEN), jnp.float32),
            [pltpu.SemaphoreType.DMA] * NBUF,
            [pltpu.SemaphoreType.DMA] * NBUF,
        ],
    )
    def k(idx_hbm, table_hbm, out_hbm, idx_v, rbuf, table_sp, gs, ws):
        wid = lax.axis_index("s") * NC + lax.axis_index("c")

        @pl.when(lax.axis_index("s") == 0)
        def _():
            pltpu.sync_copy(table_hbm, table_sp)

        pltpu.sync_copy(idx_hbm.at[wid], idx_v)
        plsc.subcore_barrier()

        def start_gather(j, b):
            for t in range(K):
                pltpu.async_copy(
                    table_sp.at[idx_v.at[j * K + t]],
                    rbuf.at[b, pl.ds(t * CHUNK, CHUNK)],
                    gs[b],
                )

        def wait_gather(b):
            for t in range(K):
                pltpu.make_async_copy(
                    table_sp, rbuf.at[b, pl.ds(t * CHUNK, CHUNK)], gs[b]
                ).wait()

        def out_slice(j):
            return out_hbm.at[pl.ds((wid * nchunks + j) * ROWS, ROWS)]

        def wait_write(j, b):
            pltpu.make_async_copy(rbuf.at[b], out_slice(j), ws[b]).wait()

        start_gather(0, 0)
        start_gather(1, 1)

        def body(i, carry):
            for u in range(NBUF):
                j = i * NBUF + u
                b = u
                b2 = (u + 2) % NBUF
                wait_gather(b)
                pltpu.async_copy(rbuf.at[b], out_slice(j), ws[b])

                @pl.when(j + 2 < nchunks)
                def _():
                    @pl.when(j >= 1)
                    def _():
                        wait_write(j - 1, b2)

                    start_gather(j + 2, b2)

            return carry

        lax.fori_loop(0, nchunks // NBUF, body, 0)
        for j in range(NBUF * (nchunks // NBUF), nchunks):
            b = j % NBUF
            wait_gather(b)
            pltpu.async_copy(rbuf.at[b], out_slice(j), ws[b])

            @pl.when(j + 2 < nchunks)
            def _():
                b2 = (j + 2) % NBUF
                wait_write(j - 1, b2)
                start_gather(j + 2, b2)

        for j in range(nchunks - NBUF, nchunks):
            wait_write(j, j % NBUF)

    return k(idx, table)


def kernel(input_ids, word_embeddings):
    b, s = input_ids.shape
    n = b * s
    assert n % (NW * CHUNK * K) == 0
    nidx = n // (NW * CHUNK)
    idx = input_ids.reshape(NW, nidx, CHUNK).astype(jnp.int32)
    out = _emb_lookup(idx, word_embeddings, nidx)
    return out.reshape(b, s, HIDDEN)
